# baseline (speedup 1.0000x reference)
import functools

import jax
import jax.numpy as jnp
from jax import lax
from jax.experimental import pallas as pl
from jax.experimental.pallas import tpu as pltpu
from jax.experimental.pallas import tpu_sc as plsc

NEG_INF = -1e30
LANES = 16


def _sc_body(T, V, LP, CH, logits_hbm, tgt_hbm, emit_hbm, sp_hbm,
             row_v, emit_v, sv_v, tgt_v, sem):
    cid = lax.axis_index("c")
    sid = lax.axis_index("s")
    w = sid * 2 + cid
    b = w // 2
    t0 = (w % 2) * (T // 2)
    nch = (T // 2) // CH
    ngr = LP // LANES

    pltpu.sync_copy(tgt_hbm.at[b], tgt_v)
    idxs = [tgt_v[pl.ds(g * LANES, LANES)] for g in range(ngr)]

    def chunk(ci, carry):
        trow = t0 + ci * CH
        pltpu.async_copy(logits_hbm.at[b, pl.ds(trow * V, CH * V)], row_v,
                         sem).wait()

        def row(r, carry2):
            def inner(j, acc):
                return acc + jnp.exp(row_v[pl.ds(r * V + j * LANES, LANES)])
            acc = lax.fori_loop(0, V // LANES, inner,
                                jnp.zeros((LANES,), jnp.float32), unroll=8)
            sv_v[r] = acc
            rbase = jnp.full((LANES,), r * V, dtype=jnp.int32)
            for g in range(ngr):
                emit_v[r, pl.ds(g * LANES, LANES)] = plsc.load_gather(
                    row_v, [rbase + idxs[g]])
            return carry2

        lax.fori_loop(0, CH, row, 0)
        pltpu.sync_copy(emit_v, emit_hbm.at[pl.ds(trow, CH), b])
        pltpu.sync_copy(sv_v, sp_hbm.at[b, pl.ds(trow, CH)])
        return carry

    lax.fori_loop(0, nch, chunk, 0)


def _rec_body(emit_ref, sp_ref, out_ref, alpha_ref, den_ref):
    jt = pl.program_id(0)
    nj = pl.num_programs(0)
    t_blk, nb, lp = emit_ref.shape

    @pl.when(jt == 0)
    def _init_den():
        den_ref[:, :] = jnp.zeros_like(den_ref)

    s = jnp.sum(sp_ref[:, :, :], axis=-1)
    den_ref[:, :] += jnp.sum(jnp.log(s)).reshape(1, 1)

    lane = lax.broadcasted_iota(jnp.int32, (nb, lp), 1)
    first = lane == 0

    def steps(E, alpha, ks):
        for k in ks:
            e_t = E[k]
            sh = jnp.concatenate(
                [jnp.full((nb, 1), NEG_INF, dtype=alpha.dtype), alpha[:, :-1]],
                axis=1)
            m = jnp.maximum(alpha, sh)
            d = -jnp.abs(alpha - sh)
            alpha = m + jnp.log1p(jnp.exp(d)) + e_t
        return alpha

    @pl.when(jt == 0)
    def _first_block():
        E = emit_ref[pl.ds(0, 8)]
        alpha = jnp.where(first, E[0], NEG_INF)
        alpha_ref[:, :] = steps(E, alpha, range(1, 8))

    tb0 = jnp.where(jt == 0, 1, 0)

    def body(tb, alpha):
        E = emit_ref[pl.ds(tb * 8, 8)]
        return steps(E, alpha, range(8))

    alpha = lax.fori_loop(tb0, t_blk // 8, body, alpha_ref[:, :])
    alpha_ref[:, :] = alpha

    @pl.when(jt == nj - 1)
    def _finish():
        num = jnp.sum(jnp.where(lane == lp - 8 - 1, alpha, 0.0))
        out_ref[:, :] = den_ref[:, :] - num.reshape(1, 1)


def kernel(logits, targets):
    B, T, V = logits.shape
    L = targets.shape[1]
    LP = L + 8
    CH = 16
    T_BLK = 160
    NJ = T // T_BLK

    tgt = jnp.pad(targets.astype(jnp.int32), ((0, 0), (0, LP - L)), mode="edge")

    mesh = plsc.VectorSubcoreMesh(core_axis_name="c", subcore_axis_name="s")
    sc = functools.partial(
        pl.kernel,
        mesh=mesh,
        compiler_params=pltpu.CompilerParams(needs_layout_passes=False),
        out_type=[
            jax.ShapeDtypeStruct((T, B, LP), jnp.float32),
            jax.ShapeDtypeStruct((B, T, LANES), jnp.float32),
        ],
        scratch_types=[
            pltpu.VMEM((CH * V,), jnp.float32),
            pltpu.VMEM((CH, LP), jnp.float32),
            pltpu.VMEM((CH, LANES), jnp.float32),
            pltpu.VMEM((LP,), jnp.int32),
            pltpu.SemaphoreType.DMA,
        ],
    )(functools.partial(_sc_body, T, V, LP, CH))
    emit, sp = sc(jnp.reshape(logits, (B, T * V)), tgt)

    out = pl.pallas_call(
        _rec_body,
        grid=(NJ,),
        in_specs=[
            pl.BlockSpec((T_BLK, B, LP), lambda jt: (jt, 0, 0)),
            pl.BlockSpec((B, T_BLK, LANES), lambda jt: (0, jt, 0)),
        ],
        out_specs=pl.BlockSpec((1, 1), lambda jt: (0, 0)),
        out_shape=jax.ShapeDtypeStruct((1, 1), jnp.float32),
        scratch_shapes=[
            pltpu.VMEM((B, LP), jnp.float32),
            pltpu.VMEM((1, 1), jnp.float32),
        ],
    )(emit, sp)
    return out[0, 0]


# speedup vs baseline: 1.3046x; 1.3046x over previous
import functools

import jax
import jax.numpy as jnp
from jax import lax
from jax.experimental import pallas as pl
from jax.experimental.pallas import tpu as pltpu
from jax.experimental.pallas import tpu_sc as plsc

NEG_INF = -1e30
LANES = 16


def _sc_body(T, V, LP, CH, logits_hbm, tgt_hbm, emit_hbm, sp_hbm,
             row_v, emit_v, sv_v, tgt_v, sem):
    cid = lax.axis_index("c")
    sid = lax.axis_index("s")
    w = sid * 2 + cid
    b = w // 2
    t0 = (w % 2) * (T // 2)
    nch = (T // 2) // CH
    ngr = LP // LANES

    pltpu.sync_copy(tgt_hbm.at[b], tgt_v)
    idxs = [tgt_v[pl.ds(g * LANES, LANES)] for g in range(ngr)]

    def chunk(ci, carry):
        trow = t0 + ci * CH
        pltpu.async_copy(logits_hbm.at[pl.ds(b * T + trow, CH)], row_v,
                         sem).wait()

        def row(r, carry2):
            def inner(j, acc):
                return acc + jnp.exp(row_v[r, pl.ds(j * LANES, LANES)])
            acc = lax.fori_loop(0, V // LANES, inner,
                                jnp.zeros((LANES,), jnp.float32), unroll=8)
            sv_v[r] = acc
            ridx = jnp.full((LANES,), r, dtype=jnp.int32)
            for g in range(ngr):
                emit_v[r, pl.ds(g * LANES, LANES)] = plsc.load_gather(
                    row_v, [ridx, idxs[g]])
            return carry2

        lax.fori_loop(0, CH, row, 0)
        pltpu.sync_copy(emit_v, emit_hbm.at[pl.ds(trow, CH), b])
        pltpu.sync_copy(sv_v, sp_hbm.at[b, pl.ds(trow, CH)])
        return carry

    lax.fori_loop(0, nch, chunk, 0)


def _rec_body(emit_ref, sp_ref, out_ref, alpha_ref, den_ref):
    jt = pl.program_id(0)
    nj = pl.num_programs(0)
    t_blk, nb, lp = emit_ref.shape

    @pl.when(jt == 0)
    def _init_den():
        den_ref[:, :] = jnp.zeros_like(den_ref)

    s = jnp.sum(sp_ref[:, :, :], axis=-1)
    den_ref[:, :] += jnp.sum(jnp.log(s)).reshape(1, 1)

    lane = lax.broadcasted_iota(jnp.int32, (nb, lp), 1)
    first = lane == 0

    def steps(E, alpha, ks):
        for k in ks:
            e_t = E[k]
            sh = jnp.concatenate(
                [jnp.full((nb, 1), NEG_INF, dtype=alpha.dtype), alpha[:, :-1]],
                axis=1)
            m = jnp.maximum(alpha, sh)
            d = -jnp.abs(alpha - sh)
            alpha = m + jnp.log1p(jnp.exp(d)) + e_t
        return alpha

    @pl.when(jt == 0)
    def _first_block():
        E = emit_ref[pl.ds(0, 8)]
        alpha = jnp.where(first, E[0], NEG_INF)
        alpha_ref[:, :] = steps(E, alpha, range(1, 8))

    tb0 = jnp.where(jt == 0, 1, 0)

    def body(tb, alpha):
        E = emit_ref[pl.ds(tb * 8, 8)]
        return steps(E, alpha, range(8))

    alpha = lax.fori_loop(tb0, t_blk // 8, body, alpha_ref[:, :])
    alpha_ref[:, :] = alpha

    @pl.when(jt == nj - 1)
    def _finish():
        num = jnp.sum(jnp.where(lane == lp - 8 - 1, alpha, 0.0))
        out_ref[:, :] = den_ref[:, :] - num.reshape(1, 1)


def kernel(logits, targets):
    B, T, V = logits.shape
    L = targets.shape[1]
    LP = L + 8
    CH = 16
    T_BLK = 160
    NJ = T // T_BLK

    tgt = jnp.pad(targets.astype(jnp.int32), ((0, 0), (0, LP - L)), mode="edge")

    mesh = plsc.VectorSubcoreMesh(core_axis_name="c", subcore_axis_name="s")
    sc = functools.partial(
        pl.kernel,
        mesh=mesh,
        compiler_params=pltpu.CompilerParams(needs_layout_passes=False),
        out_type=[
            jax.ShapeDtypeStruct((T, B, LP), jnp.float32),
            jax.ShapeDtypeStruct((B, T, LANES), jnp.float32),
        ],
        scratch_types=[
            pltpu.VMEM((CH, V), jnp.float32),
            pltpu.VMEM((CH, LP), jnp.float32),
            pltpu.VMEM((CH, LANES), jnp.float32),
            pltpu.VMEM((LP,), jnp.int32),
            pltpu.SemaphoreType.DMA,
        ],
    )(functools.partial(_sc_body, T, V, LP, CH))
    emit, sp = sc(jnp.reshape(logits, (B * T, V)), tgt)

    out = pl.pallas_call(
        _rec_body,
        grid=(NJ,),
        in_specs=[
            pl.BlockSpec((T_BLK, B, LP), lambda jt: (jt, 0, 0)),
            pl.BlockSpec((B, T_BLK, LANES), lambda jt: (0, jt, 0)),
        ],
        out_specs=pl.BlockSpec((1, 1), lambda jt: (0, 0)),
        out_shape=jax.ShapeDtypeStruct((1, 1), jnp.float32),
        scratch_shapes=[
            pltpu.VMEM((B, LP), jnp.float32),
            pltpu.VMEM((1, 1), jnp.float32),
        ],
    )(emit, sp)
    return out[0, 0]


# speedup vs baseline: 1.9348x; 1.4830x over previous
import functools

import jax
import jax.numpy as jnp
from jax import lax
from jax.experimental import pallas as pl
from jax.experimental.pallas import tpu as pltpu
from jax.experimental.pallas import tpu_sc as plsc

NEG_INF = -1e30
LANES = 16


def _sc_body(T, V, LP, CH, logits_hbm, tgt_hbm, emit_hbm, sp_hbm,
             row_v0, row_v1, emit_v, sv_v, tgt_v, sem0, sem1):
    cid = lax.axis_index("c")
    sid = lax.axis_index("s")
    w = sid * 2 + cid
    b = w // 2
    t0 = (w % 2) * (T // 2)
    nch = (T // 2) // CH
    ngr = LP // LANES
    wide = 8

    pltpu.sync_copy(tgt_hbm.at[b], tgt_v)
    idxs = [tgt_v[pl.ds(g * LANES, LANES)] for g in range(ngr)]

    def start(ci, buf, sem):
        pltpu.async_copy(logits_hbm.at[pl.ds(b * T + t0 + ci * CH, CH)],
                         buf, sem)

    def wait(buf, sem):
        pltpu.make_async_copy(logits_hbm.at[pl.ds(0, CH)], buf, sem).wait()

    def process(ci, buf):
        trow = t0 + ci * CH

        def row(r, carry2):
            def inner(j, accs):
                return tuple(
                    accs[u] + jnp.exp(buf[r, pl.ds((j * wide + u) * LANES,
                                                   LANES)])
                    for u in range(wide))
            accs = lax.fori_loop(
                0, V // (LANES * wide), inner,
                tuple(jnp.zeros((LANES,), jnp.float32) for _ in range(wide)))
            while len(accs) > 1:
                accs = tuple(accs[2 * u] + accs[2 * u + 1]
                             for u in range(len(accs) // 2))
            sv_v[r] = accs[0]
            ridx = jnp.full((LANES,), r, dtype=jnp.int32)
            for g in range(ngr):
                emit_v[r, pl.ds(g * LANES, LANES)] = plsc.load_gather(
                    buf, [ridx, idxs[g]])
            return carry2

        lax.fori_loop(0, CH, row, 0)
        pltpu.sync_copy(emit_v, emit_hbm.at[pl.ds(trow, CH), b])
        pltpu.sync_copy(sv_v, sp_hbm.at[b, pl.ds(trow, CH)])

    start(0, row_v0, sem0)

    def chunk2(i, carry):
        wait(row_v0, sem0)
        start(2 * i + 1, row_v1, sem1)
        process(2 * i, row_v0)
        wait(row_v1, sem1)

        @pl.when(2 * i + 2 < nch)
        def _next():
            start(2 * i + 2, row_v0, sem0)

        process(2 * i + 1, row_v1)
        return carry

    lax.fori_loop(0, nch // 2, chunk2, 0)


def _rec_body(emit_ref, sp_ref, out_ref, alpha_ref, den_ref):
    jt = pl.program_id(0)
    nj = pl.num_programs(0)
    t_blk, nb, lp = emit_ref.shape

    @pl.when(jt == 0)
    def _init_den():
        den_ref[:, :] = jnp.zeros_like(den_ref)

    s = jnp.sum(sp_ref[:, :, :], axis=-1)
    den_ref[:, :] += jnp.sum(jnp.log(s)).reshape(1, 1)

    lane = lax.broadcasted_iota(jnp.int32, (nb, lp), 1)
    first = lane == 0

    def steps(E, alpha, ks):
        for k in ks:
            e_t = E[k]
            sh = jnp.concatenate(
                [jnp.full((nb, 1), NEG_INF, dtype=alpha.dtype), alpha[:, :-1]],
                axis=1)
            m = jnp.maximum(alpha, sh)
            d = -jnp.abs(alpha - sh)
            alpha = m + jnp.log1p(jnp.exp(d)) + e_t
        return alpha

    @pl.when(jt == 0)
    def _first_block():
        E = emit_ref[pl.ds(0, 8)]
        alpha = jnp.where(first, E[0], NEG_INF)
        alpha_ref[:, :] = steps(E, alpha, range(1, 8))

    tb0 = jnp.where(jt == 0, 1, 0)

    def body(tb, alpha):
        E = emit_ref[pl.ds(tb * 8, 8)]
        return steps(E, alpha, range(8))

    alpha = lax.fori_loop(tb0, t_blk // 8, body, alpha_ref[:, :])
    alpha_ref[:, :] = alpha

    @pl.when(jt == nj - 1)
    def _finish():
        num = jnp.sum(jnp.where(lane == lp - 8 - 1, alpha, 0.0))
        out_ref[:, :] = den_ref[:, :] - num.reshape(1, 1)


def kernel(logits, targets):
    B, T, V = logits.shape
    L = targets.shape[1]
    LP = L + 8
    CH = 16
    T_BLK = 160
    NJ = T // T_BLK

    tgt = jnp.pad(targets.astype(jnp.int32), ((0, 0), (0, LP - L)), mode="edge")

    mesh = plsc.VectorSubcoreMesh(core_axis_name="c", subcore_axis_name="s")
    sc = functools.partial(
        pl.kernel,
        mesh=mesh,
        compiler_params=pltpu.CompilerParams(needs_layout_passes=False),
        out_type=[
            jax.ShapeDtypeStruct((T, B, LP), jnp.float32),
            jax.ShapeDtypeStruct((B, T, LANES), jnp.float32),
        ],
        scratch_types=[
            pltpu.VMEM((CH, V), jnp.float32),
            pltpu.VMEM((CH, V), jnp.float32),
            pltpu.VMEM((CH, LP), jnp.float32),
            pltpu.VMEM((CH, LANES), jnp.float32),
            pltpu.VMEM((LP,), jnp.int32),
            pltpu.SemaphoreType.DMA,
            pltpu.SemaphoreType.DMA,
        ],
    )(functools.partial(_sc_body, T, V, LP, CH))
    emit, sp = sc(jnp.reshape(logits, (B * T, V)), tgt)

    out = pl.pallas_call(
        _rec_body,
        grid=(NJ,),
        in_specs=[
            pl.BlockSpec((T_BLK, B, LP), lambda jt: (jt, 0, 0)),
            pl.BlockSpec((B, T_BLK, LANES), lambda jt: (0, jt, 0)),
        ],
        out_specs=pl.BlockSpec((1, 1), lambda jt: (0, 0)),
        out_shape=jax.ShapeDtypeStruct((1, 1), jnp.float32),
        scratch_shapes=[
            pltpu.VMEM((B, LP), jnp.float32),
            pltpu.VMEM((1, 1), jnp.float32),
        ],
    )(emit, sp)
    return out[0, 0]


# speedup vs baseline: 2.2733x; 1.1750x over previous
import functools

import jax
import jax.numpy as jnp
from jax import lax
from jax.experimental import pallas as pl
from jax.experimental.pallas import tpu as pltpu
from jax.experimental.pallas import tpu_sc as plsc

NEG_INF = -1e30
LANES = 16


def _sc_body(T, V, LP, CH, logits_hbm, tgt_hbm, emit_hbm, sp_hbm,
             row_v0, row_v1, emit_v, sv_v, tgt_v, sem0, sem1):
    cid = lax.axis_index("c")
    sid = lax.axis_index("s")
    w = sid * 2 + cid
    b = w // 2
    t0 = (w % 2) * (T // 2)
    nch = (T // 2) // CH
    ngr = LP // LANES
    wide = 8

    pltpu.sync_copy(tgt_hbm.at[b], tgt_v)
    idxs = [tgt_v[pl.ds(g * LANES, LANES)] for g in range(ngr)]

    def start(ci, buf, sem):
        pltpu.async_copy(logits_hbm.at[pl.ds(b * T + t0 + ci * CH, CH)],
                         buf, sem)

    def wait(buf, sem):
        pltpu.make_async_copy(logits_hbm.at[pl.ds(0, CH)], buf, sem).wait()

    def process(ci, buf):
        trow = t0 + ci * CH

        def row(r, carry2):
            def inner(j, accs):
                return tuple(
                    accs[u] + jnp.exp(buf[r, pl.ds((j * wide + u) * LANES,
                                                   LANES)])
                    for u in range(wide))
            accs = lax.fori_loop(
                0, V // (LANES * wide), inner,
                tuple(jnp.zeros((LANES,), jnp.float32) for _ in range(wide)))
            while len(accs) > 1:
                accs = tuple(accs[2 * u] + accs[2 * u + 1]
                             for u in range(len(accs) // 2))
            sv_v[r] = accs[0]
            ridx = jnp.full((LANES,), r, dtype=jnp.int32)
            for g in range(ngr):
                emit_v[r, pl.ds(g * LANES, LANES)] = plsc.load_gather(
                    buf, [ridx, idxs[g]])
            return carry2

        lax.fori_loop(0, CH, row, 0)
        pltpu.sync_copy(emit_v, emit_hbm.at[pl.ds(trow, CH), b])
        pltpu.sync_copy(sv_v, sp_hbm.at[b, pl.ds(trow, CH)])

    start(0, row_v0, sem0)

    def chunk2(i, carry):
        wait(row_v0, sem0)
        start(2 * i + 1, row_v1, sem1)
        process(2 * i, row_v0)
        wait(row_v1, sem1)

        @pl.when(2 * i + 2 < nch)
        def _next():
            start(2 * i + 2, row_v0, sem0)

        process(2 * i + 1, row_v1)
        return carry

    lax.fori_loop(0, nch // 2, chunk2, 0)


def _rec_body(emit_ref, sp_ref, out_ref, alpha_ref, den_ref,
              w0_ref, w1_ref, w2_ref):
    jt = pl.program_id(0)
    nj = pl.num_programs(0)
    t_blk, nb, lp = emit_ref.shape
    npair = t_blk // 2

    @pl.when(jt == 0)
    def _init_den():
        den_ref[:, :] = jnp.zeros_like(den_ref)

    s = jnp.sum(sp_ref[:, :, :], axis=-1)
    den_ref[:, :] += jnp.sum(jnp.log(s)).reshape(1, 1)

    lane = lax.broadcasted_iota(jnp.int32, (nb, lp), 1)
    first = lane == 0

    def pre(c, carry):
        E = emit_ref[pl.ds(c * 16, 16)].reshape(8, 2, nb, lp)
        e1, e2 = E[:, 0], E[:, 1]
        sh_e1 = jnp.concatenate(
            [jnp.full((8, nb, 1), NEG_INF, jnp.float32), e1[:, :, :-1]], axis=2)
        m = jnp.maximum(e1, sh_e1)
        d = -jnp.abs(e1 - sh_e1)
        w0_ref[pl.ds(c * 8, 8)] = e1 + e2
        w1_ref[pl.ds(c * 8, 8)] = m + jnp.log1p(jnp.exp(d)) + e2
        w2_ref[pl.ds(c * 8, 8)] = sh_e1 + e2
        return carry

    lax.fori_loop(0, t_blk // 16, pre, 0)

    def plain_step(e_t, alpha):
        sh = jnp.concatenate(
            [jnp.full((nb, 1), NEG_INF, dtype=alpha.dtype), alpha[:, :-1]],
            axis=1)
        m = jnp.maximum(alpha, sh)
        d = -jnp.abs(alpha - sh)
        return m + jnp.log1p(jnp.exp(d)) + e_t

    @pl.when(jt == 0)
    def _first_block():
        alpha = jnp.where(first, emit_ref[0], NEG_INF)
        alpha_ref[:, :] = plain_step(emit_ref[1], alpha)

    p0 = jnp.where(jt == 0, 1, 0)

    def pair_step(p, alpha):
        sh = jnp.concatenate(
            [jnp.full((nb, 1), NEG_INF, jnp.float32), alpha[:, :-1]], axis=1)
        sh2 = jnp.concatenate(
            [jnp.full((nb, 2), NEG_INF, jnp.float32), alpha[:, :-2]], axis=1)
        x0 = alpha + w0_ref[p]
        x1 = sh + w1_ref[p]
        x2 = sh2 + w2_ref[p]
        m = jnp.maximum(jnp.maximum(x0, x1), x2)
        ssum = jnp.exp(x0 - m) + jnp.exp(x1 - m) + jnp.exp(x2 - m)
        return m + jnp.log(ssum)

    alpha = lax.fori_loop(p0, npair, pair_step, alpha_ref[:, :])
    alpha_ref[:, :] = alpha

    @pl.when(jt == nj - 1)
    def _finish():
        num = jnp.sum(jnp.where(lane == lp - 8 - 1, alpha, 0.0))
        out_ref[:, :] = den_ref[:, :] - num.reshape(1, 1)


def kernel(logits, targets):
    B, T, V = logits.shape
    L = targets.shape[1]
    LP = L + 8
    CH = 16
    T_BLK = 160
    NJ = T // T_BLK

    tgt = jnp.pad(targets.astype(jnp.int32), ((0, 0), (0, LP - L)), mode="edge")

    mesh = plsc.VectorSubcoreMesh(core_axis_name="c", subcore_axis_name="s")
    sc = functools.partial(
        pl.kernel,
        mesh=mesh,
        compiler_params=pltpu.CompilerParams(needs_layout_passes=False),
        out_type=[
            jax.ShapeDtypeStruct((T, B, LP), jnp.float32),
            jax.ShapeDtypeStruct((B, T, LANES), jnp.float32),
        ],
        scratch_types=[
            pltpu.VMEM((CH, V), jnp.float32),
            pltpu.VMEM((CH, V), jnp.float32),
            pltpu.VMEM((CH, LP), jnp.float32),
            pltpu.VMEM((CH, LANES), jnp.float32),
            pltpu.VMEM((LP,), jnp.int32),
            pltpu.SemaphoreType.DMA,
            pltpu.SemaphoreType.DMA,
        ],
    )(functools.partial(_sc_body, T, V, LP, CH))
    emit, sp = sc(jnp.reshape(logits, (B * T, V)), tgt)

    out = pl.pallas_call(
        _rec_body,
        grid=(NJ,),
        in_specs=[
            pl.BlockSpec((T_BLK, B, LP), lambda jt: (jt, 0, 0)),
            pl.BlockSpec((B, T_BLK, LANES), lambda jt: (0, jt, 0)),
        ],
        out_specs=pl.BlockSpec((1, 1), lambda jt: (0, 0)),
        out_shape=jax.ShapeDtypeStruct((1, 1), jnp.float32),
        scratch_shapes=[
            pltpu.VMEM((B, LP), jnp.float32),
            pltpu.VMEM((1, 1), jnp.float32),
            pltpu.VMEM((T_BLK // 2, B, LP), jnp.float32),
            pltpu.VMEM((T_BLK // 2, B, LP), jnp.float32),
            pltpu.VMEM((T_BLK // 2, B, LP), jnp.float32),
        ],
    )(emit, sp)
    return out[0, 0]


# speedup vs baseline: 2.3989x; 1.0552x over previous
import functools

import jax
import jax.numpy as jnp
from jax import lax
from jax.experimental import pallas as pl
from jax.experimental.pallas import tpu as pltpu
from jax.experimental.pallas import tpu_sc as plsc

NEG_INF = -1e30
LANES = 16


def _sc_body(T, V, LP, CH, logits_hbm, tgt_hbm, emit_hbm, sp_hbm,
             row_v0, row_v1, emit_v, sv_v, tgt_v, sem0, sem1):
    cid = lax.axis_index("c")
    sid = lax.axis_index("s")
    w = sid * 2 + cid
    b = w // 2
    t0 = (w % 2) * (T // 2)
    nch = (T // 2) // CH
    ngr = LP // LANES
    wide = 8

    pltpu.sync_copy(tgt_hbm.at[b], tgt_v)
    idxs = [tgt_v[pl.ds(g * LANES, LANES)] for g in range(ngr)]

    def start(ci, buf, sem):
        pltpu.async_copy(logits_hbm.at[pl.ds(b * T + t0 + ci * CH, CH)],
                         buf, sem)

    def wait(buf, sem):
        pltpu.make_async_copy(logits_hbm.at[pl.ds(0, CH)], buf, sem).wait()

    def process(ci, buf):
        trow = t0 + ci * CH

        def row(r, carry2):
            def inner(j, accs):
                return tuple(
                    accs[u] + jnp.exp(buf[r, pl.ds((j * wide + u) * LANES,
                                                   LANES)])
                    for u in range(wide))
            accs = lax.fori_loop(
                0, V // (LANES * wide), inner,
                tuple(jnp.zeros((LANES,), jnp.float32) for _ in range(wide)))
            while len(accs) > 1:
                accs = tuple(accs[2 * u] + accs[2 * u + 1]
                             for u in range(len(accs) // 2))
            sv_v[r] = accs[0]
            ridx = jnp.full((LANES,), r, dtype=jnp.int32)
            for g in range(ngr):
                emit_v[r, pl.ds(g * LANES, LANES)] = plsc.load_gather(
                    buf, [ridx, idxs[g]])
            return carry2

        lax.fori_loop(0, CH, row, 0)
        pltpu.sync_copy(emit_v, emit_hbm.at[pl.ds(trow, CH), b])
        pltpu.sync_copy(sv_v, sp_hbm.at[b, pl.ds(trow, CH)])

    start(0, row_v0, sem0)

    def chunk2(i, carry):
        wait(row_v0, sem0)
        start(2 * i + 1, row_v1, sem1)
        process(2 * i, row_v0)
        wait(row_v1, sem1)

        @pl.when(2 * i + 2 < nch)
        def _next():
            start(2 * i + 2, row_v0, sem0)

        process(2 * i + 1, row_v1)
        return carry

    lax.fori_loop(0, nch // 2, chunk2, 0)


def _rec_body(emit_ref, sp_ref, out_ref, alpha_ref, den_ref,
              w0_ref, w1_ref, w2_ref, c0_ref, c1_ref, c2_ref, c3_ref, c4_ref):
    jt = pl.program_id(0)
    nj = pl.num_programs(0)
    t_blk, nb, lp = emit_ref.shape
    npair = t_blk // 2

    @pl.when(jt == 0)
    def _init_den():
        den_ref[:, :] = jnp.zeros_like(den_ref)

    s = jnp.sum(sp_ref[:, :, :], axis=-1)
    den_ref[:, :] += jnp.sum(jnp.log(s)).reshape(1, 1)

    lane = lax.broadcasted_iota(jnp.int32, (nb, lp), 1)
    first = lane == 0

    def pre(c, carry):
        E = emit_ref[pl.ds(c * 16, 16)].reshape(8, 2, nb, lp)
        e1, e2 = E[:, 0], E[:, 1]
        sh_e1 = jnp.concatenate(
            [jnp.full((8, nb, 1), NEG_INF, jnp.float32), e1[:, :, :-1]], axis=2)
        m = jnp.maximum(e1, sh_e1)
        d = -jnp.abs(e1 - sh_e1)
        w0_ref[pl.ds(c * 8, 8)] = e1 + e2
        w1_ref[pl.ds(c * 8, 8)] = m + jnp.log1p(jnp.exp(d)) + e2
        w2_ref[pl.ds(c * 8, 8)] = sh_e1 + e2
        return carry

    lax.fori_loop(0, t_blk // 16, pre, 0)

    def shift(x, j):
        return jnp.concatenate(
            [jnp.full(x.shape[:-1] + (j,), NEG_INF, jnp.float32),
             x[..., :-j]], axis=-1)

    def lse2(a, b):
        m = jnp.maximum(a, b)
        return m + jnp.log(jnp.exp(a - m) + jnp.exp(b - m))

    def lse3(a, b, c):
        m = jnp.maximum(jnp.maximum(a, b), c)
        return m + jnp.log(jnp.exp(a - m) + jnp.exp(b - m) + jnp.exp(c - m))

    def pre2(c, carry):
        A0 = w0_ref[pl.ds(c * 8, 8)].reshape(4, 2, nb, lp)
        A1 = w1_ref[pl.ds(c * 8, 8)].reshape(4, 2, nb, lp)
        A2 = w2_ref[pl.ds(c * 8, 8)].reshape(4, 2, nb, lp)
        a0, b0 = A0[:, 0], A0[:, 1]
        a1, b1 = A1[:, 0], A1[:, 1]
        a2, b2 = A2[:, 0], A2[:, 1]
        c0_ref[pl.ds(c * 4, 4)] = a0 + b0
        c1_ref[pl.ds(c * 4, 4)] = lse2(a1 + b0, shift(a0, 1) + b1)
        c2_ref[pl.ds(c * 4, 4)] = lse3(a2 + b0, shift(a1, 1) + b1,
                                       shift(a0, 2) + b2)
        c3_ref[pl.ds(c * 4, 4)] = lse2(shift(a2, 1) + b1, shift(a1, 2) + b2)
        c4_ref[pl.ds(c * 4, 4)] = shift(a2, 2) + b2
        return carry

    lax.fori_loop(0, t_blk // 16, pre2, 0)

    def plain_step(e_t, alpha):
        sh = jnp.concatenate(
            [jnp.full((nb, 1), NEG_INF, dtype=alpha.dtype), alpha[:, :-1]],
            axis=1)
        m = jnp.maximum(alpha, sh)
        d = -jnp.abs(alpha - sh)
        return m + jnp.log1p(jnp.exp(d)) + e_t

    @pl.when(jt == 0)
    def _first_block():
        alpha = jnp.where(first, emit_ref[0], NEG_INF)
        alpha_ref[:, :] = plain_step(emit_ref[1], alpha)

    def pair_step(p, alpha):
        x0 = alpha + w0_ref[p]
        x1 = shift(alpha, 1) + w1_ref[p]
        x2 = shift(alpha, 2) + w2_ref[p]
        m = jnp.maximum(jnp.maximum(x0, x1), x2)
        ssum = jnp.exp(x0 - m) + jnp.exp(x1 - m) + jnp.exp(x2 - m)
        return m + jnp.log(ssum)

    @pl.when(jt == 0)
    def _first_pair():
        alpha_ref[:, :] = pair_step(1, alpha_ref[:, :])

    q0 = jnp.where(jt == 0, 1, 0)

    def quad_step(q, alpha):
        x0 = alpha + c0_ref[q]
        x1 = shift(alpha, 1) + c1_ref[q]
        x2 = shift(alpha, 2) + c2_ref[q]
        x3 = shift(alpha, 3) + c3_ref[q]
        x4 = shift(alpha, 4) + c4_ref[q]
        m = jnp.maximum(jnp.maximum(jnp.maximum(x0, x1), jnp.maximum(x2, x3)),
                        x4)
        ssum = (jnp.exp(x0 - m) + jnp.exp(x1 - m) + jnp.exp(x2 - m)
                + jnp.exp(x3 - m) + jnp.exp(x4 - m))
        return m + jnp.log(ssum)

    alpha = lax.fori_loop(q0, t_blk // 4, quad_step, alpha_ref[:, :])
    alpha_ref[:, :] = alpha

    @pl.when(jt == nj - 1)
    def _finish():
        num = jnp.sum(jnp.where(lane == lp - 8 - 1, alpha, 0.0))
        out_ref[:, :] = den_ref[:, :] - num.reshape(1, 1)


def kernel(logits, targets):
    B, T, V = logits.shape
    L = targets.shape[1]
    LP = L + 8
    CH = 16
    T_BLK = 160
    NJ = T // T_BLK

    tgt = jnp.pad(targets.astype(jnp.int32), ((0, 0), (0, LP - L)), mode="edge")

    mesh = plsc.VectorSubcoreMesh(core_axis_name="c", subcore_axis_name="s")
    sc = functools.partial(
        pl.kernel,
        mesh=mesh,
        compiler_params=pltpu.CompilerParams(needs_layout_passes=False),
        out_type=[
            jax.ShapeDtypeStruct((T, B, LP), jnp.float32),
            jax.ShapeDtypeStruct((B, T, LANES), jnp.float32),
        ],
        scratch_types=[
            pltpu.VMEM((CH, V), jnp.float32),
            pltpu.VMEM((CH, V), jnp.float32),
            pltpu.VMEM((CH, LP), jnp.float32),
            pltpu.VMEM((CH, LANES), jnp.float32),
            pltpu.VMEM((LP,), jnp.int32),
            pltpu.SemaphoreType.DMA,
            pltpu.SemaphoreType.DMA,
        ],
    )(functools.partial(_sc_body, T, V, LP, CH))
    emit, sp = sc(jnp.reshape(logits, (B * T, V)), tgt)

    out = pl.pallas_call(
        _rec_body,
        grid=(NJ,),
        in_specs=[
            pl.BlockSpec((T_BLK, B, LP), lambda jt: (jt, 0, 0)),
            pl.BlockSpec((B, T_BLK, LANES), lambda jt: (0, jt, 0)),
        ],
        out_specs=pl.BlockSpec((1, 1), lambda jt: (0, 0)),
        out_shape=jax.ShapeDtypeStruct((1, 1), jnp.float32),
        scratch_shapes=[
            pltpu.VMEM((B, LP), jnp.float32),
            pltpu.VMEM((1, 1), jnp.float32),
            pltpu.VMEM((T_BLK // 2, B, LP), jnp.float32),
            pltpu.VMEM((T_BLK // 2, B, LP), jnp.float32),
            pltpu.VMEM((T_BLK // 2, B, LP), jnp.float32),
            pltpu.VMEM((T_BLK // 4, B, LP), jnp.float32),
            pltpu.VMEM((T_BLK // 4, B, LP), jnp.float32),
            pltpu.VMEM((T_BLK // 4, B, LP), jnp.float32),
            pltpu.VMEM((T_BLK // 4, B, LP), jnp.float32),
            pltpu.VMEM((T_BLK // 4, B, LP), jnp.float32),
        ],
    )(emit, sp)
    return out[0, 0]


# speedup vs baseline: 2.4494x; 1.0211x over previous
import functools

import jax
import jax.numpy as jnp
from jax import lax
from jax.experimental import pallas as pl
from jax.experimental.pallas import tpu as pltpu
from jax.experimental.pallas import tpu_sc as plsc

NEG_INF = -1e30
LANES = 16


def _sc_body(T, V, LP, CH, logits_hbm, tgt_hbm, emit_hbm, sp_hbm,
             row_v0, row_v1, emit_v, sv_v, tgt_v, sem0, sem1):
    cid = lax.axis_index("c")
    sid = lax.axis_index("s")
    w = sid * 2 + cid
    b = w // 2
    t0 = (w % 2) * (T // 2)
    nch = (T // 2) // CH
    ngr = LP // LANES
    wide = 8

    pltpu.sync_copy(tgt_hbm.at[b], tgt_v)
    idxs = [tgt_v[pl.ds(g * LANES, LANES)] for g in range(ngr)]

    def start(ci, buf, sem):
        pltpu.async_copy(logits_hbm.at[pl.ds(b * T + t0 + ci * CH, CH)],
                         buf, sem)

    def wait(buf, sem):
        pltpu.make_async_copy(logits_hbm.at[pl.ds(0, CH)], buf, sem).wait()

    def process(ci, buf):
        trow = t0 + ci * CH

        def row(r, carry2):
            accs = [jnp.exp(buf[r, pl.ds(u * LANES, LANES)])
                    for u in range(wide)]
            for j in range(1, V // (LANES * wide)):
                for u in range(wide):
                    accs[u] = accs[u] + jnp.exp(
                        buf[r, pl.ds((j * wide + u) * LANES, LANES)])
            accs = tuple(accs)
            while len(accs) > 1:
                accs = tuple(accs[2 * u] + accs[2 * u + 1]
                             for u in range(len(accs) // 2))
            sv_v[r] = accs[0]
            ridx = jnp.full((LANES,), r, dtype=jnp.int32)
            for g in range(ngr):
                emit_v[r, pl.ds(g * LANES, LANES)] = plsc.load_gather(
                    buf, [ridx, idxs[g]])
            return carry2

        lax.fori_loop(0, CH, row, 0)
        pltpu.sync_copy(emit_v, emit_hbm.at[pl.ds(trow, CH), b])
        pltpu.sync_copy(sv_v, sp_hbm.at[b, pl.ds(trow, CH)])

    start(0, row_v0, sem0)

    def chunk2(i, carry):
        wait(row_v0, sem0)
        start(2 * i + 1, row_v1, sem1)
        process(2 * i, row_v0)
        wait(row_v1, sem1)

        @pl.when(2 * i + 2 < nch)
        def _next():
            start(2 * i + 2, row_v0, sem0)

        process(2 * i + 1, row_v1)
        return carry

    lax.fori_loop(0, nch // 2, chunk2, 0)


def _rec_body(emit_ref, sp_ref, out_ref, alpha_ref, den_ref,
              w0_ref, w1_ref, w2_ref, c0_ref, c1_ref, c2_ref, c3_ref, c4_ref):
    jt = pl.program_id(0)
    nj = pl.num_programs(0)
    t_blk, nb, lp = emit_ref.shape
    npair = t_blk // 2

    @pl.when(jt == 0)
    def _init_den():
        den_ref[:, :] = jnp.zeros_like(den_ref)

    s = jnp.sum(sp_ref[:, :, :], axis=-1)
    den_ref[:, :] += jnp.sum(jnp.log(s)).reshape(1, 1)

    lane = lax.broadcasted_iota(jnp.int32, (nb, lp), 1)
    first = lane == 0

    def pre(c, carry):
        E = emit_ref[pl.ds(c * 16, 16)].reshape(8, 2, nb, lp)
        e1, e2 = E[:, 0], E[:, 1]
        sh_e1 = jnp.concatenate(
            [jnp.full((8, nb, 1), NEG_INF, jnp.float32), e1[:, :, :-1]], axis=2)
        m = jnp.maximum(e1, sh_e1)
        d = -jnp.abs(e1 - sh_e1)
        w0_ref[pl.ds(c * 8, 8)] = e1 + e2
        w1_ref[pl.ds(c * 8, 8)] = m + jnp.log1p(jnp.exp(d)) + e2
        w2_ref[pl.ds(c * 8, 8)] = sh_e1 + e2
        return carry

    lax.fori_loop(0, t_blk // 16, pre, 0)

    def shift(x, j):
        return jnp.concatenate(
            [jnp.full(x.shape[:-1] + (j,), NEG_INF, jnp.float32),
             x[..., :-j]], axis=-1)

    def lse2(a, b):
        m = jnp.maximum(a, b)
        return m + jnp.log(jnp.exp(a - m) + jnp.exp(b - m))

    def lse3(a, b, c):
        m = jnp.maximum(jnp.maximum(a, b), c)
        return m + jnp.log(jnp.exp(a - m) + jnp.exp(b - m) + jnp.exp(c - m))

    def pre2(c, carry):
        A0 = w0_ref[pl.ds(c * 8, 8)].reshape(4, 2, nb, lp)
        A1 = w1_ref[pl.ds(c * 8, 8)].reshape(4, 2, nb, lp)
        A2 = w2_ref[pl.ds(c * 8, 8)].reshape(4, 2, nb, lp)
        a0, b0 = A0[:, 0], A0[:, 1]
        a1, b1 = A1[:, 0], A1[:, 1]
        a2, b2 = A2[:, 0], A2[:, 1]
        c0_ref[pl.ds(c * 4, 4)] = a0 + b0
        c1_ref[pl.ds(c * 4, 4)] = lse2(a1 + b0, shift(a0, 1) + b1)
        c2_ref[pl.ds(c * 4, 4)] = lse3(a2 + b0, shift(a1, 1) + b1,
                                       shift(a0, 2) + b2)
        c3_ref[pl.ds(c * 4, 4)] = lse2(shift(a2, 1) + b1, shift(a1, 2) + b2)
        c4_ref[pl.ds(c * 4, 4)] = shift(a2, 2) + b2
        return carry

    lax.fori_loop(0, t_blk // 16, pre2, 0)

    def plain_step(e_t, alpha):
        sh = jnp.concatenate(
            [jnp.full((nb, 1), NEG_INF, dtype=alpha.dtype), alpha[:, :-1]],
            axis=1)
        m = jnp.maximum(alpha, sh)
        d = -jnp.abs(alpha - sh)
        return m + jnp.log1p(jnp.exp(d)) + e_t

    @pl.when(jt == 0)
    def _first_block():
        alpha = jnp.where(first, emit_ref[0], NEG_INF)
        alpha_ref[:, :] = plain_step(emit_ref[1], alpha)

    def pair_step(p, alpha):
        x0 = alpha + w0_ref[p]
        x1 = shift(alpha, 1) + w1_ref[p]
        x2 = shift(alpha, 2) + w2_ref[p]
        m = jnp.maximum(jnp.maximum(x0, x1), x2)
        ssum = jnp.exp(x0 - m) + jnp.exp(x1 - m) + jnp.exp(x2 - m)
        return m + jnp.log(ssum)

    @pl.when(jt == 0)
    def _first_pair():
        alpha_ref[:, :] = pair_step(1, alpha_ref[:, :])

    q0 = jnp.where(jt == 0, 1, 0)

    def quad_step(q, alpha):
        x0 = alpha + c0_ref[q]
        x1 = shift(alpha, 1) + c1_ref[q]
        x2 = shift(alpha, 2) + c2_ref[q]
        x3 = shift(alpha, 3) + c3_ref[q]
        x4 = shift(alpha, 4) + c4_ref[q]
        m = jnp.maximum(jnp.maximum(jnp.maximum(x0, x1), jnp.maximum(x2, x3)),
                        x4)
        ssum = (jnp.exp(x0 - m) + jnp.exp(x1 - m) + jnp.exp(x2 - m)
                + jnp.exp(x3 - m) + jnp.exp(x4 - m))
        return m + jnp.log(ssum)

    alpha = lax.fori_loop(q0, t_blk // 4, quad_step, alpha_ref[:, :])
    alpha_ref[:, :] = alpha

    @pl.when(jt == nj - 1)
    def _finish():
        num = jnp.sum(jnp.where(lane == lp - 8 - 1, alpha, 0.0))
        out_ref[:, :] = den_ref[:, :] - num.reshape(1, 1)


def kernel(logits, targets):
    B, T, V = logits.shape
    L = targets.shape[1]
    LP = L + 8
    CH = 16
    T_BLK = 160
    NJ = T // T_BLK

    tgt = jnp.pad(targets.astype(jnp.int32), ((0, 0), (0, LP - L)), mode="edge")

    mesh = plsc.VectorSubcoreMesh(core_axis_name="c", subcore_axis_name="s")
    sc = functools.partial(
        pl.kernel,
        mesh=mesh,
        compiler_params=pltpu.CompilerParams(needs_layout_passes=False),
        out_type=[
            jax.ShapeDtypeStruct((T, B, LP), jnp.float32),
            jax.ShapeDtypeStruct((B, T, LANES), jnp.float32),
        ],
        scratch_types=[
            pltpu.VMEM((CH, V), jnp.float32),
            pltpu.VMEM((CH, V), jnp.float32),
            pltpu.VMEM((CH, LP), jnp.float32),
            pltpu.VMEM((CH, LANES), jnp.float32),
            pltpu.VMEM((LP,), jnp.int32),
            pltpu.SemaphoreType.DMA,
            pltpu.SemaphoreType.DMA,
        ],
    )(functools.partial(_sc_body, T, V, LP, CH))
    emit, sp = sc(jnp.reshape(logits, (B * T, V)), tgt)

    out = pl.pallas_call(
        _rec_body,
        grid=(NJ,),
        in_specs=[
            pl.BlockSpec((T_BLK, B, LP), lambda jt: (jt, 0, 0)),
            pl.BlockSpec((B, T_BLK, LANES), lambda jt: (0, jt, 0)),
        ],
        out_specs=pl.BlockSpec((1, 1), lambda jt: (0, 0)),
        out_shape=jax.ShapeDtypeStruct((1, 1), jnp.float32),
        scratch_shapes=[
            pltpu.VMEM((B, LP), jnp.float32),
            pltpu.VMEM((1, 1), jnp.float32),
            pltpu.VMEM((T_BLK // 2, B, LP), jnp.float32),
            pltpu.VMEM((T_BLK // 2, B, LP), jnp.float32),
            pltpu.VMEM((T_BLK // 2, B, LP), jnp.float32),
            pltpu.VMEM((T_BLK // 4, B, LP), jnp.float32),
            pltpu.VMEM((T_BLK // 4, B, LP), jnp.float32),
            pltpu.VMEM((T_BLK // 4, B, LP), jnp.float32),
            pltpu.VMEM((T_BLK // 4, B, LP), jnp.float32),
            pltpu.VMEM((T_BLK // 4, B, LP), jnp.float32),
        ],
    )(emit, sp)
    return out[0, 0]
